# Initial kernel scaffold; baseline (speedup 1.0000x reference)
import jax
import jax.numpy as jnp
from jax.experimental import pallas as pl


def kernel(coordinates, adjacency, node_features, edge_features, joint_types, params):
    raise NotImplementedError("write your pallas kernel here")


# speedup vs baseline: 1.5708x; 1.5708x over previous
import jax
import jax.numpy as jnp
from jax.experimental import pallas as pl

N = 128
M = 128
IB = 32

_NEG = -1e9


def _mm(x, w):
    return jnp.dot(x, w, preferred_element_type=jnp.float32)


def _edge_conv(x, e_aug, wa, ba, wb, bb, use_e):
    d = x.shape[1]
    wx = wa[0:d]
    wd = wa[d:2 * d]
    a = _mm(x, wx - wd) + ba
    bj = _mm(x, wd)
    outs = []
    for t in range(N // IB):
        sl = slice(t * IB, (t + 1) * IB)
        l1 = a[sl][:, None, :] + bj[None, :, :]
        if use_e:
            we = wa[2 * d:]
            eb = e_aug[sl, :, 0:16].reshape(IB * N, 16)
            l1 = l1 + _mm(eb, we).reshape(IB, N, M)
        l1 = jnp.maximum(l1, 0.0)
        l2 = _mm(l1.reshape(IB * N, M), wb) + bb
        l2 = jnp.maximum(l2, 0.0).reshape(IB, N, M)
        l2 = l2 + e_aug[sl, :, 16:17]
        outs.append(jnp.max(l2, axis=1))
    return jnp.concatenate(outs, axis=0)


def _body(coord_ref, node_ref, edge_ref, joint_ref,
          w1, b1, w2, b2,
          wa_e1, ba_e1, wb_e1, bb_e1,
          wa_e2, ba_e2, wb_e2, bb_e2,
          w3, b3,
          wa_c1, ba_c1, wb_c1, bb_c1,
          wa_c2, ba_c2, wb_c2, bb_c2,
          wa_c3, ba_c3, wb_c3, bb_c3,
          wo1, bo1, wo2, bo2,
          y1_ref, y2_ref):
    coord = coord_ref[0]
    node = node_ref[0]
    e_aug = edge_ref[0]
    joint = joint_ref[0]

    x = jnp.maximum(_mm(coord, w1[...]) + b1[...], 0.0)
    x = jnp.maximum(_mm(x, w2[...]) + b2[...], 0.0)
    x = jnp.concatenate([x, node, joint], axis=1)

    x = _edge_conv(x, e_aug, wa_e1[...], ba_e1[...], wb_e1[...], bb_e1[...], True)
    x = _edge_conv(x, e_aug, wa_e2[...], ba_e2[...], wb_e2[...], bb_e2[...], True)

    g = jnp.max(x, axis=0, keepdims=True)
    w3v = w3[...]
    x = jnp.maximum(_mm(x, w3v[0:M]) + _mm(g, w3v[M:2 * M]) + b3[...], 0.0)

    x = _edge_conv(x, e_aug, wa_c1[...], ba_c1[...], wb_c1[...], bb_c1[...], False)
    ec1 = x
    x = _edge_conv(x, e_aug, wa_c2[...], ba_c2[...], wb_c2[...], bb_c2[...], False)
    ec2 = x
    x = x + ec1
    x = _edge_conv(x, e_aug, wa_c3[...], ba_c3[...], wb_c3[...], bb_c3[...], False)
    x = x + ec2

    wo1v, bo1v = wo1[...], bo1[...]
    y1 = jnp.maximum(_mm(x, wo1v) + bo1v, 0.0)
    y1 = jnp.maximum(_mm(y1, wo1v) + bo1v, 0.0)
    wo2v, bo2v = wo2[...], bo2[...]
    y2 = jnp.maximum(_mm(x, wo2v) + bo2v, 0.0)
    y2 = jnp.maximum(_mm(y2, wo2v) + bo2v, 0.0)
    y1_ref[0] = y1
    y2_ref[0] = y2


def kernel(coordinates, adjacency, node_features, edge_features, joint_types, params):
    B = coordinates.shape[0]
    f32 = jnp.float32

    coords = jnp.pad(coordinates, ((0, 0), (0, 0), (0, 8 - coordinates.shape[-1])))
    madd = jnp.where(adjacency > 0, 0.0, -1e9).astype(f32)
    e_aug = jnp.concatenate([edge_features, madd[..., None]], axis=-1)

    def wb(name, pad_rows=0):
        W, b = params[name]
        if pad_rows:
            W = jnp.pad(W, ((0, pad_rows), (0, 0)))
        return W, b.reshape(1, -1)

    weight_list = []
    for name, pad in (('h1', 8 - 3), ('h2', 0),
                      ('ece1_a', 0), ('ece1_b', 0),
                      ('ece2_a', 0), ('ece2_b', 0),
                      ('h3', 0),
                      ('ec1_a', 0), ('ec1_b', 0),
                      ('ec2_a', 0), ('ec2_b', 0),
                      ('ec3_a', 0), ('ec3_b', 0),
                      ('out1', 0), ('out2', 0)):
        W, b = wb(name, pad)
        weight_list.append(W)
        weight_list.append(b)

    data = [coords, node_features, e_aug, joint_types]

    def data_spec(arr):
        blk = (1,) + arr.shape[1:]
        nd = len(blk)
        return pl.BlockSpec(blk, lambda b, _nd=nd: (b,) + (0,) * (_nd - 1))

    def w_spec(arr):
        nd = arr.ndim
        return pl.BlockSpec(arr.shape, lambda b, _nd=nd: (0,) * _nd)

    in_specs = [data_spec(a) for a in data] + [w_spec(w) for w in weight_list]
    out_specs = [pl.BlockSpec((1, N, M), lambda b: (b, 0, 0)),
                 pl.BlockSpec((1, N, M), lambda b: (b, 0, 0))]
    out_shape = [jax.ShapeDtypeStruct((B, N, M), f32),
                 jax.ShapeDtypeStruct((B, N, M), f32)]

    y1, y2 = pl.pallas_call(
        _body,
        grid=(B,),
        in_specs=in_specs,
        out_specs=out_specs,
        out_shape=out_shape,
    )(*data, *weight_list)
    return (y1, y2)
